# Initial kernel scaffold; baseline (speedup 1.0000x reference)
import functools

import jax
import jax.numpy as jnp
import numpy as np
from jax import lax
from jax.experimental import pallas as pl
from jax.experimental.pallas import tpu as pltpu
from jax.experimental.pallas import tpu_sc as plsc

_VOCAB = 100000
_D = 1024
_B = 4
_L = 4096

_NC = 2
_NS = 16
_NW = _NC * _NS
_LPW = _L // _NW
_C = 32
_NCHUNK = _LPW // _C


def _pe_table() -> np.ndarray:
    pos = np.arange(_L, dtype=np.float32)[:, None]
    dim = np.arange(_D, dtype=np.float32)
    pe = np.zeros((_L, _D), dtype=np.float32)
    pe[:, 0::2] = np.sin(pos / 10000.0 ** (dim[0::2] / _D)).astype(np.float32)
    pe[:, 1::2] = np.cos(pos / 10000.0 ** (dim[1::2] / _D)).astype(np.float32)
    return pe


_PE = _pe_table()


def _body(x_ref, pe_ref, table_ref, out_ref,
          idx_v, pe_v, rows0, rows1,
          gsem0, gsem1, osem0, osem1):
    rows = (rows0, rows1)
    gsem = (gsem0, gsem1)
    osem = (osem0, osem1)

    wid = lax.axis_index("s") * _NC + lax.axis_index("c")
    l0 = wid * _LPW

    for b in range(_B):
        pltpu.sync_copy(x_ref.at[pl.ds(b * _L + l0, _LPW)], idx_v.at[b])

    def issue_gather(t):
        c, b = divmod(t, _B)
        idx = idx_v.at[b, pl.ds(c * _C, _C)]
        return pltpu.async_copy(table_ref.at[idx], rows[t % 2], gsem[t % 2])

    def issue_scatter(t):
        c, b = divmod(t, _B)
        row0 = b * _L + l0 + c * _C
        return pltpu.async_copy(rows[t % 2], out_ref.at[pl.ds(row0, _C)],
                                osem[t % 2])

    def add_pe(rv):
        @pl.loop(0, _C * (_D // 16), unroll=8)
        def _add(i):
            r = i >> 6
            col = (i & 63) * 16
            sl = pl.ds(col, 16)
            rv[r, sl] = rv[r, sl] + pe_v[r, sl]

    nt = _NCHUNK * _B
    descs = {}
    descs[("g", 0)] = issue_gather(0)
    for t in range(nt):
        c, b = divmod(t, _B)
        if t + 1 < nt:
            if t >= 1:
                descs[("o", t - 1)].wait()
            descs[("g", t + 1)] = issue_gather(t + 1)
        if b == 0:
            pltpu.sync_copy(pe_ref.at[pl.ds(l0 + c * _C, _C)], pe_v)
        descs[("g", t)].wait()
        add_pe(rows[t % 2])
        descs[("o", t)] = issue_scatter(t)
    descs[("o", nt - 2)].wait()
    descs[("o", nt - 1)].wait()


@functools.partial(
    pl.kernel,
    out_type=jax.ShapeDtypeStruct((_B * _L, _D), jnp.float32),
    mesh=plsc.VectorSubcoreMesh(core_axis_name="c", subcore_axis_name="s"),
    scratch_types=[
        pltpu.VMEM((_B, _LPW), jnp.int32),
        pltpu.VMEM((_C, _D), jnp.float32),
        pltpu.VMEM((_C, _D), jnp.float32),
        pltpu.VMEM((_C, _D), jnp.float32),
        pltpu.SemaphoreType.DMA,
        pltpu.SemaphoreType.DMA,
        pltpu.SemaphoreType.DMA,
        pltpu.SemaphoreType.DMA,
    ],
)
def _embed_sc(x_ref, pe_ref, table_ref, out_ref, *scratch):
    _body(x_ref, pe_ref, table_ref, out_ref, *scratch)


def kernel(x, table):
    pe = jnp.asarray(_PE)
    x_flat = x.reshape(_B * _L).astype(jnp.int32)
    out = _embed_sc(x_flat, pe, table)
    return out.reshape(_B, _L, _D)


# speedup vs baseline: 2.0121x; 2.0121x over previous
import functools

import jax
import jax.numpy as jnp
import numpy as np
from jax import lax
from jax.experimental import pallas as pl
from jax.experimental.pallas import tpu as pltpu
from jax.experimental.pallas import tpu_sc as plsc

_VOCAB = 100000
_D = 1024
_B = 4
_L = 4096

_NC = 2
_NS = 16
_NW = _NC * _NS
_LPW = _L // _NW
_C = 32
_NCHUNK = _LPW // _C


def _pe_table() -> np.ndarray:
    pos = np.arange(_L, dtype=np.float32)[:, None]
    dim = np.arange(_D, dtype=np.float32)
    pe = np.zeros((_L, _D), dtype=np.float32)
    pe[:, 0::2] = np.sin(pos / 10000.0 ** (dim[0::2] / _D)).astype(np.float32)
    pe[:, 1::2] = np.cos(pos / 10000.0 ** (dim[1::2] / _D)).astype(np.float32)
    return pe


_PE = _pe_table()


def _body(x_ref, pe_ref, table_ref, out_ref,
          idx_v, pe_v, rows0, rows1,
          gsem0, gsem1, osem0, osem1):
    rows = (rows0, rows1)
    gsem = (gsem0, gsem1)
    osem = (osem0, osem1)

    wid = lax.axis_index("s") * _NC + lax.axis_index("c")
    l0 = wid * _LPW

    for b in range(_B):
        pltpu.sync_copy(x_ref.at[pl.ds(b * _L + l0, _LPW)], idx_v.at[b])

    def issue_gather(t):
        c, b = divmod(t, _B)
        idx = idx_v.at[b, pl.ds(c * _C, _C)]
        return pltpu.async_copy(table_ref.at[idx], rows[t % 2], gsem[t % 2])

    def issue_scatter(t):
        c, b = divmod(t, _B)
        row0 = b * _L + l0 + c * _C
        return pltpu.async_copy(rows[t % 2], out_ref.at[pl.ds(row0, _C)],
                                osem[t % 2])

    def add_pe(rv):
        @pl.loop(0, _C * (_D // 16), unroll=8)
        def _add(i):
            r = i >> 6
            col = (i & 63) * 16
            sl = pl.ds(col, 16)
            rv[r, sl] = rv[r, sl] + pe_v[r, sl]

    nt = _NCHUNK * _B
    descs = {}
    descs[("g", 0)] = issue_gather(0)
    for t in range(nt):
        c, b = divmod(t, _B)
        if t + 1 < nt:
            if t >= 1:
                descs[("o", t - 1)].wait()
            descs[("g", t + 1)] = issue_gather(t + 1)
        if b == 0:
            pltpu.sync_copy(pe_ref.at[pl.ds(l0 + c * _C, _C)], pe_v)
        descs[("g", t)].wait()
        add_pe(rows[t % 2])
        descs[("o", t)] = issue_scatter(t)
    descs[("o", nt - 2)].wait()
    descs[("o", nt - 1)].wait()


@functools.lru_cache(maxsize=1)
def _build():
    return pl.kernel(
        _body,
        out_type=jax.ShapeDtypeStruct((_B * _L, _D), jnp.float32),
        mesh=plsc.VectorSubcoreMesh(core_axis_name="c", subcore_axis_name="s",
                                    num_cores=_NC, num_subcores=_NS),
        scratch_types=[
            pltpu.VMEM((_B, _LPW), jnp.int32),
            pltpu.VMEM((_C, _D), jnp.float32),
            pltpu.VMEM((_C, _D), jnp.float32),
            pltpu.VMEM((_C, _D), jnp.float32),
            pltpu.SemaphoreType.DMA,
            pltpu.SemaphoreType.DMA,
            pltpu.SemaphoreType.DMA,
            pltpu.SemaphoreType.DMA,
        ],
    )


def kernel(x, table):
    pe = jnp.asarray(_PE)
    x_flat = x.reshape(_B * _L).astype(jnp.int32)
    out = _build()(x_flat, pe, table)
    return out.reshape(_B, _L, _D)


# speedup vs baseline: 2.3277x; 1.1569x over previous
import functools

import jax
import jax.numpy as jnp
import numpy as np
from jax import lax
from jax.experimental import pallas as pl
from jax.experimental.pallas import tpu as pltpu
from jax.experimental.pallas import tpu_sc as plsc

_VOCAB = 100000
_D = 1024
_B = 4
_L = 4096

_NC = 2
_NS = 16
_NW = _NC * _NS
_LPW = _L // _NW
_C = 16
_NCHUNK = _LPW // _C


def _pe_table() -> np.ndarray:
    pos = np.arange(_L, dtype=np.float32)[:, None]
    dim = np.arange(_D, dtype=np.float32)
    pe = np.zeros((_L, _D), dtype=np.float32)
    pe[:, 0::2] = np.sin(pos / 10000.0 ** (dim[0::2] / _D)).astype(np.float32)
    pe[:, 1::2] = np.cos(pos / 10000.0 ** (dim[1::2] / _D)).astype(np.float32)
    return pe


_PE = _pe_table()


def _body(x_ref, pe_ref, table_ref, out_ref,
          idx_v, pe0, pe1, r0, r1, r2,
          gs0, gs1, gs2, os0, os1, os2, ps0, ps1):
    rows = (r0, r1, r2)
    gsem = (gs0, gs1, gs2)
    osem = (os0, os1, os2)
    pe_v = (pe0, pe1)
    psem = (ps0, ps1)

    wid = lax.axis_index("s") * _NC + lax.axis_index("c")
    l0 = wid * _LPW

    for b in range(_B):
        pltpu.sync_copy(x_ref.at[pl.ds(b * _L + l0, _LPW)], idx_v.at[b])

    def issue_pe(c):
        return pltpu.async_copy(pe_ref.at[pl.ds(l0 + c * _C, _C)],
                                pe_v[c % 2], psem[c % 2])

    def issue_gather(t):
        c, b = divmod(t, _B)
        idx = idx_v.at[b, pl.ds(c * _C, _C)]
        return pltpu.async_copy(table_ref.at[idx], rows[t % 3], gsem[t % 3])

    def issue_scatter(t):
        c, b = divmod(t, _B)
        row0 = b * _L + l0 + c * _C
        return pltpu.async_copy(rows[t % 3], out_ref.at[pl.ds(row0, _C)],
                                osem[t % 3])

    def add_pe(rv, pv):
        @pl.loop(0, _C * (_D // 16), unroll=8)
        def _add(i):
            r = i >> 6
            sl = pl.ds((i & 63) * 16, 16)
            plsc.addupdate(rv.at[r, sl], pv[r, sl])

    nt = _NCHUNK * _B
    descs = {}
    descs[("p", 0)] = issue_pe(0)
    descs[("g", 0)] = issue_gather(0)
    for t in range(nt):
        c, b = divmod(t, _B)
        if t + 1 < nt:
            if t >= 2:
                descs[("o", t - 2)].wait()
            descs[("g", t + 1)] = issue_gather(t + 1)
        if b == 0:
            if c + 1 < _NCHUNK:
                descs[("p", c + 1)] = issue_pe(c + 1)
            descs[("p", c)].wait()
        descs[("g", t)].wait()
        add_pe(rows[t % 3], pe_v[c % 2])
        descs[("o", t)] = issue_scatter(t)
    descs[("o", nt - 3)].wait()
    descs[("o", nt - 2)].wait()
    descs[("o", nt - 1)].wait()


@functools.lru_cache(maxsize=1)
def _build():
    return pl.kernel(
        _body,
        out_type=jax.ShapeDtypeStruct((_B * _L, _D), jnp.float32),
        mesh=plsc.VectorSubcoreMesh(core_axis_name="c", subcore_axis_name="s",
                                    num_cores=_NC, num_subcores=_NS),
        scratch_types=[
            pltpu.VMEM((_B, _LPW), jnp.int32),
            pltpu.VMEM((_C, _D), jnp.float32),
            pltpu.VMEM((_C, _D), jnp.float32),
            pltpu.VMEM((_C, _D), jnp.float32),
            pltpu.VMEM((_C, _D), jnp.float32),
            pltpu.VMEM((_C, _D), jnp.float32),
            pltpu.SemaphoreType.DMA,
            pltpu.SemaphoreType.DMA,
            pltpu.SemaphoreType.DMA,
            pltpu.SemaphoreType.DMA,
            pltpu.SemaphoreType.DMA,
            pltpu.SemaphoreType.DMA,
            pltpu.SemaphoreType.DMA,
            pltpu.SemaphoreType.DMA,
        ],
    )


def kernel(x, table):
    pe = jnp.asarray(_PE)
    x_flat = x.reshape(_B * _L).astype(jnp.int32)
    out = _build()(x_flat, pe, table)
    return out.reshape(_B, _L, _D)
